# baseline (speedup 1.0000x reference)
import functools

import jax
import jax.numpy as jnp
from jax.experimental import pallas as pl
from jax.experimental.pallas import tpu as pltpu
from jax.experimental.pallas import tpu_sc as plsc

_GC = 256
_WORKERS = 32
_NB = 200


def _gelu(x):
    return x * 0.5 * (1.0 + jax.lax.erf(x * 0.7071067811865476))


def _ln(x, g, b):
    m = jnp.mean(x, axis=-1, keepdims=True)
    d = x - m
    v = jnp.mean(d * d, axis=-1, keepdims=True)
    return d * jax.lax.rsqrt(v + 1e-5) * g + b


def _sc_gather(table, idx1d):
    Mp = idx1d.shape[0]
    D = table.shape[1]
    idx2d = idx1d.reshape(1, Mp)
    mesh = plsc.VectorSubcoreMesh(core_axis_name="c", subcore_axis_name="s")

    @functools.partial(
        pl.kernel,
        out_type=jax.ShapeDtypeStruct((Mp, D), table.dtype),
        mesh=mesh,
    )
    def run(x_hbm, i_hbm, o_hbm):
        def body(i_vmem, o_vmem):
            pltpu.sync_copy(x_hbm.at[i_vmem.at[0]], o_vmem)

        pltpu.emit_pipeline(
            body,
            grid=(Mp // _GC,),
            in_specs=[pl.BlockSpec((1, _GC), lambda i: (0, i))],
            out_specs=[pl.BlockSpec((_GC, D), lambda i: (i, 0))],
            core_axis_name=("c", "s"),
            dimension_semantics=(pltpu.PARALLEL,),
        )(i_hbm, o_hbm)

    return run(table, idx2d)


def _bdot(a, b):
    return jnp.dot(a.astype(jnp.bfloat16), b.astype(jnp.bfloat16),
                   preferred_element_type=jnp.float32)


def _proj_body(hv, w1a, b1, w1c, pre1_o, g1_o):
    x = hv[...]
    pre1_o[...] = (
        jnp.dot(x, w1a[...], preferred_element_type=jnp.float32) + b1[...]
    )
    g1_o[...] = jnp.dot(x, w1c[...], preferred_element_type=jnp.float32)


def _block1_body(hv, pre1, he, g1, w1b, w2, b2, w3, b3, win, bi, wout, bo,
                 n1g, n1b, n2g, n2b, w11a, b11, w11c,
                 hv2_o, pre11_o, g11_o):
    nb, Hd = hv.shape
    ne = he.shape[0]
    K = ne // nb
    e = _bdot(he[...], w1b[...])
    t = (e + g1[...].astype(jnp.float32)).reshape(nb, K, Hd) + pre1[...][:, None, :]
    t = _gelu(t).reshape(ne, Hd)
    u = _gelu(_bdot(t, w2[...]) + b2[...])
    msg = _bdot(u, w3[...]) + b3[...]
    dh = msg.reshape(nb, K, Hd).sum(axis=1) * (1.0 / 30.0)
    h = _ln(hv[...] + dh, n1g[...], n1b[...])
    f = _gelu(jnp.dot(h, win[...], preferred_element_type=jnp.float32) + bi[...])
    f = jnp.dot(f, wout[...], preferred_element_type=jnp.float32) + bo[...]
    y = _ln(h + f, n2g[...], n2b[...])
    hv2_o[...] = y
    pre11_o[...] = (
        jnp.dot(y, w11a[...], preferred_element_type=jnp.float32) + b11[...]
    )
    g11_o[...] = jnp.dot(y, w11c[...], preferred_element_type=jnp.float32)


def _block2_body(he, g2, pre11, w11b, w12, b12, w13, b13, n3g, n3b, heo):
    ne, Hd = he.shape
    nb = pre11.shape[0]
    K = ne // nb
    e = _bdot(he[...], w11b[...])
    t = (e + g2[...].astype(jnp.float32)).reshape(nb, K, Hd) + pre11[...][:, None, :]
    t = _gelu(t).reshape(ne, Hd)
    u = _gelu(_bdot(t, w12[...]) + b12[...])
    msg = _bdot(u, w13[...]) + b13[...]
    heo[...] = _ln(he[...] + msg, n3g[...], n3b[...])


def _row_spec(rows, cols):
    return pl.BlockSpec((rows, cols), lambda i: (i, 0))


def _const_spec(rows, cols):
    return pl.BlockSpec((rows, cols), lambda i: (0, 0))


def kernel(h_V, h_E, E_idx, params):
    B, N, Hd = h_V.shape
    K = h_E.shape[2]
    M = N * K
    hv = h_V.reshape(N, Hd)
    he = h_E.reshape(M, Hd)
    idx = E_idx.reshape(-1).astype(jnp.int32)

    p = params
    w1 = p["W1"]["w"]
    w11 = p["W11"]["w"]
    w1a, w1b, w1c = w1[:Hd], w1[Hd:2 * Hd], w1[2 * Hd:]
    w11a, w11b, w11c = w11[:Hd], w11[Hd:2 * Hd], w11[2 * Hd:]
    b1 = p["W1"]["b"].reshape(1, Hd)
    b11 = p["W11"]["b"].reshape(1, Hd)
    b2 = p["W2"]["b"].reshape(1, Hd)
    b3 = p["W3"]["b"].reshape(1, Hd)
    b12 = p["W12"]["b"].reshape(1, Hd)
    b13 = p["W13"]["b"].reshape(1, Hd)
    bi = p["Win"]["b"].reshape(1, -1)
    bo = p["Wout"]["b"].reshape(1, Hd)
    n1g = p["norm1"]["g"].reshape(1, Hd)
    n1b = p["norm1"]["b"].reshape(1, Hd)
    n2g = p["norm2"]["g"].reshape(1, Hd)
    n2b = p["norm2"]["b"].reshape(1, Hd)
    n3g = p["norm3"]["g"].reshape(1, Hd)
    n3b = p["norm3"]["b"].reshape(1, Hd)
    Hi = p["Win"]["w"].shape[1]

    align = _GC * _WORKERS
    Mp = ((M + align - 1) // align) * align
    if Mp != M:
        idx = jnp.concatenate([idx, jnp.zeros((Mp - M,), jnp.int32)])

    pre1, g1t = pl.pallas_call(
        _proj_body,
        out_shape=[
            jax.ShapeDtypeStruct((N, Hd), jnp.float32),
            jax.ShapeDtypeStruct((N, Hd), jnp.float32),
        ],
    )(hv, w1a, b1, w1c)

    grid = (N // _NB,)
    ne = _NB * K

    g1 = _sc_gather(g1t, idx)[:M]

    hv2, pre11, g11t = pl.pallas_call(
        _block1_body,
        grid=grid,
        in_specs=[
            _row_spec(_NB, Hd),
            _row_spec(_NB, Hd),
            _row_spec(ne, Hd),
            _row_spec(ne, Hd),
            _const_spec(Hd, Hd),
            _const_spec(Hd, Hd),
            _const_spec(1, Hd),
            _const_spec(Hd, Hd),
            _const_spec(1, Hd),
            _const_spec(Hd, Hi),
            _const_spec(1, Hi),
            _const_spec(Hi, Hd),
            _const_spec(1, Hd),
            _const_spec(1, Hd),
            _const_spec(1, Hd),
            _const_spec(1, Hd),
            _const_spec(1, Hd),
            _const_spec(Hd, Hd),
            _const_spec(1, Hd),
            _const_spec(Hd, Hd),
        ],
        out_specs=[
            _row_spec(_NB, Hd),
            _row_spec(_NB, Hd),
            _row_spec(_NB, Hd),
        ],
        out_shape=[
            jax.ShapeDtypeStruct((N, Hd), jnp.float32),
            jax.ShapeDtypeStruct((N, Hd), jnp.float32),
            jax.ShapeDtypeStruct((N, Hd), jnp.float32),
        ],
    )(hv, pre1, he, g1, w1b, p["W2"]["w"], b2, p["W3"]["w"], b3,
      p["Win"]["w"], bi, p["Wout"]["w"], bo, n1g, n1b, n2g, n2b,
      w11a, b11, w11c)

    g2 = _sc_gather(g11t, idx)[:M]

    heo = pl.pallas_call(
        _block2_body,
        grid=grid,
        in_specs=[
            _row_spec(ne, Hd),
            _row_spec(ne, Hd),
            _row_spec(_NB, Hd),
            _const_spec(Hd, Hd),
            _const_spec(Hd, Hd),
            _const_spec(1, Hd),
            _const_spec(Hd, Hd),
            _const_spec(1, Hd),
            _const_spec(1, Hd),
            _const_spec(1, Hd),
        ],
        out_specs=[_row_spec(ne, Hd)],
        out_shape=[jax.ShapeDtypeStruct((M, Hd), jnp.float32)],
    )(he, g2, pre11, w11b, p["W12"]["w"], b12, p["W13"]["w"], b13,
      n3g, n3b)[0]

    return hv2.reshape(B, N, Hd), heo.reshape(B, N, K, Hd)


# speedup vs baseline: 1.4512x; 1.4512x over previous
import functools

import jax
import jax.numpy as jnp
from jax.experimental import pallas as pl
from jax.experimental.pallas import tpu as pltpu
from jax.experimental.pallas import tpu_sc as plsc

_GC = 128
_WORKERS = 32
_NB = 200


def _gelu(x):
    return x * 0.5 * (1.0 + jax.lax.erf(x * 0.7071067811865476))


def _ln(x, g, b):
    m = jnp.mean(x, axis=-1, keepdims=True)
    d = x - m
    v = jnp.mean(d * d, axis=-1, keepdims=True)
    return d * jax.lax.rsqrt(v + 1e-5) * g + b


def _sc_gather(table, idx1d):
    Mp = idx1d.shape[0]
    D = table.shape[1]
    idx2d = idx1d.reshape(1, Mp)
    mesh = plsc.VectorSubcoreMesh(core_axis_name="c", subcore_axis_name="s")

    @functools.partial(
        pl.kernel,
        out_type=jax.ShapeDtypeStruct((Mp, D), table.dtype),
        mesh=mesh,
    )
    def run(x_hbm, i_hbm, o_hbm):
        def body(i_vmem, o_vmem):
            pltpu.sync_copy(x_hbm.at[i_vmem.at[0]], o_vmem)

        pltpu.emit_pipeline(
            body,
            grid=(Mp // _GC,),
            in_specs=[pl.BlockSpec((1, _GC), lambda i: (0, i))],
            out_specs=[pl.BlockSpec((_GC, D), lambda i: (i, 0))],
            core_axis_name=("c", "s"),
            dimension_semantics=(pltpu.PARALLEL,),
        )(i_hbm, o_hbm)

    return run(table, idx2d)


def _bdot(a, b):
    return jnp.dot(a.astype(jnp.bfloat16), b.astype(jnp.bfloat16),
                   preferred_element_type=jnp.float32)


def _proj_body(hv, w1a, b1, w1c, pre1_o, g1_o):
    x = hv[...]
    pre1_o[...] = (
        jnp.dot(x, w1a[...], preferred_element_type=jnp.float32) + b1[...]
    )
    g1_o[...] = jnp.dot(x, w1c[...], preferred_element_type=jnp.float32)


def _block1_body(hv, pre1, he, g1, w1b, w2, b2, w3, b3, win, bi, wout, bo,
                 n1g, n1b, n2g, n2b, w11a, b11, w11c,
                 hv2_o, pre11_o, g11_o):
    nb, Hd = hv.shape
    ne = he.shape[0]
    K = ne // nb
    e = _bdot(he[...], w1b[...])
    t = (e + g1[...].astype(jnp.float32)).reshape(nb, K, Hd) + pre1[...][:, None, :]
    t = _gelu(t).reshape(ne, Hd)
    u = _gelu(_bdot(t, w2[...]) + b2[...])
    msg = _bdot(u, w3[...]) + b3[...]
    dh = msg.reshape(nb, K, Hd).sum(axis=1) * (1.0 / 30.0)
    h = _ln(hv[...] + dh, n1g[...], n1b[...])
    f = _gelu(jnp.dot(h, win[...], preferred_element_type=jnp.float32) + bi[...])
    f = jnp.dot(f, wout[...], preferred_element_type=jnp.float32) + bo[...]
    y = _ln(h + f, n2g[...], n2b[...])
    hv2_o[...] = y
    pre11_o[...] = (
        jnp.dot(y, w11a[...], preferred_element_type=jnp.float32) + b11[...]
    )
    g11_o[...] = jnp.dot(y, w11c[...], preferred_element_type=jnp.float32)


def _block2_body(he, g2, pre11, w11b, w12, b12, w13, b13, n3g, n3b, heo):
    ne, Hd = he.shape
    nb = pre11.shape[0]
    K = ne // nb
    e = _bdot(he[...], w11b[...])
    t = (e + g2[...].astype(jnp.float32)).reshape(nb, K, Hd) + pre11[...][:, None, :]
    t = _gelu(t).reshape(ne, Hd)
    u = _gelu(_bdot(t, w12[...]) + b12[...])
    msg = _bdot(u, w13[...]) + b13[...]
    heo[...] = _ln(he[...] + msg, n3g[...], n3b[...])


def _row_spec(rows, cols):
    return pl.BlockSpec((rows, cols), lambda i: (i, 0))


def _const_spec(rows, cols):
    return pl.BlockSpec((rows, cols), lambda i: (0, 0))


def kernel(h_V, h_E, E_idx, params):
    B, N, Hd = h_V.shape
    K = h_E.shape[2]
    M = N * K
    hv = h_V.reshape(N, Hd)
    he = h_E.reshape(M, Hd)
    idx = E_idx.reshape(-1).astype(jnp.int32)

    p = params
    w1 = p["W1"]["w"]
    w11 = p["W11"]["w"]
    w1a, w1b, w1c = w1[:Hd], w1[Hd:2 * Hd], w1[2 * Hd:]
    w11a, w11b, w11c = w11[:Hd], w11[Hd:2 * Hd], w11[2 * Hd:]
    b1 = p["W1"]["b"].reshape(1, Hd)
    b11 = p["W11"]["b"].reshape(1, Hd)
    b2 = p["W2"]["b"].reshape(1, Hd)
    b3 = p["W3"]["b"].reshape(1, Hd)
    b12 = p["W12"]["b"].reshape(1, Hd)
    b13 = p["W13"]["b"].reshape(1, Hd)
    bi = p["Win"]["b"].reshape(1, -1)
    bo = p["Wout"]["b"].reshape(1, Hd)
    n1g = p["norm1"]["g"].reshape(1, Hd)
    n1b = p["norm1"]["b"].reshape(1, Hd)
    n2g = p["norm2"]["g"].reshape(1, Hd)
    n2b = p["norm2"]["b"].reshape(1, Hd)
    n3g = p["norm3"]["g"].reshape(1, Hd)
    n3b = p["norm3"]["b"].reshape(1, Hd)
    Hi = p["Win"]["w"].shape[1]

    align = _GC * _WORKERS
    Mp = ((M + align - 1) // align) * align
    if Mp != M:
        idx = jnp.concatenate([idx, jnp.zeros((Mp - M,), jnp.int32)])

    pre1, g1t = pl.pallas_call(
        _proj_body,
        out_shape=[
            jax.ShapeDtypeStruct((N, Hd), jnp.float32),
            jax.ShapeDtypeStruct((N, Hd), jnp.float32),
        ],
    )(hv, w1a, b1, w1c)

    grid = (N // _NB,)
    ne = _NB * K

    g1 = _sc_gather(g1t, idx)

    hv2, pre11, g11t = pl.pallas_call(
        _block1_body,
        grid=grid,
        in_specs=[
            _row_spec(_NB, Hd),
            _row_spec(_NB, Hd),
            _row_spec(ne, Hd),
            _row_spec(ne, Hd),
            _const_spec(Hd, Hd),
            _const_spec(Hd, Hd),
            _const_spec(1, Hd),
            _const_spec(Hd, Hd),
            _const_spec(1, Hd),
            _const_spec(Hd, Hi),
            _const_spec(1, Hi),
            _const_spec(Hi, Hd),
            _const_spec(1, Hd),
            _const_spec(1, Hd),
            _const_spec(1, Hd),
            _const_spec(1, Hd),
            _const_spec(1, Hd),
            _const_spec(Hd, Hd),
            _const_spec(1, Hd),
            _const_spec(Hd, Hd),
        ],
        out_specs=[
            _row_spec(_NB, Hd),
            _row_spec(_NB, Hd),
            _row_spec(_NB, Hd),
        ],
        out_shape=[
            jax.ShapeDtypeStruct((N, Hd), jnp.float32),
            jax.ShapeDtypeStruct((N, Hd), jnp.float32),
            jax.ShapeDtypeStruct((N, Hd), jnp.float32),
        ],
    )(hv, pre1, he, g1, w1b, p["W2"]["w"], b2, p["W3"]["w"], b3,
      p["Win"]["w"], bi, p["Wout"]["w"], bo, n1g, n1b, n2g, n2b,
      w11a, b11, w11c)

    g2 = _sc_gather(g11t, idx)

    heo = pl.pallas_call(
        _block2_body,
        grid=grid,
        in_specs=[
            _row_spec(ne, Hd),
            _row_spec(ne, Hd),
            _row_spec(_NB, Hd),
            _const_spec(Hd, Hd),
            _const_spec(Hd, Hd),
            _const_spec(1, Hd),
            _const_spec(Hd, Hd),
            _const_spec(1, Hd),
            _const_spec(1, Hd),
            _const_spec(1, Hd),
        ],
        out_specs=[_row_spec(ne, Hd)],
        out_shape=[jax.ShapeDtypeStruct((M, Hd), jnp.float32)],
    )(he, g2, pre11, w11b, p["W12"]["w"], b12, p["W13"]["w"], b13,
      n3g, n3b)[0]

    return hv2.reshape(B, N, Hd), heo.reshape(B, N, K, Hd)


# speedup vs baseline: 1.5630x; 1.0770x over previous
import functools

import jax
import jax.numpy as jnp
from jax.experimental import pallas as pl
from jax.experimental.pallas import tpu as pltpu
from jax.experimental.pallas import tpu_sc as plsc

_GC = 128
_WORKERS = 32
_NB = 200


def _gelu(x):
    return x * 0.5 * (1.0 + jax.lax.erf(x * 0.7071067811865476))


def _ln(x, g, b):
    m = jnp.mean(x, axis=-1, keepdims=True)
    d = x - m
    v = jnp.mean(d * d, axis=-1, keepdims=True)
    return d * jax.lax.rsqrt(v + 1e-5) * g + b


def _sc_gather(table, idx1d):
    Mp = idx1d.shape[0]
    D = table.shape[1]
    idx2d = idx1d.reshape(1, Mp)
    mesh = plsc.VectorSubcoreMesh(core_axis_name="c", subcore_axis_name="s")

    @functools.partial(
        pl.kernel,
        out_type=jax.ShapeDtypeStruct((Mp, D), table.dtype),
        mesh=mesh,
    )
    def run(x_hbm, i_hbm, o_hbm):
        def body(i_vmem, o_vmem):
            pltpu.sync_copy(x_hbm.at[i_vmem.at[0]], o_vmem)

        pltpu.emit_pipeline(
            body,
            grid=(Mp // _GC,),
            in_specs=[pl.BlockSpec((1, _GC), lambda i: (0, i))],
            out_specs=[pl.BlockSpec((_GC, D), lambda i: (i, 0))],
            core_axis_name=("c", "s"),
            dimension_semantics=(pltpu.PARALLEL,),
        )(i_hbm, o_hbm)

    return run(table, idx2d)


def _bdot(a, b):
    return jnp.dot(a.astype(jnp.bfloat16), b.astype(jnp.bfloat16),
                   preferred_element_type=jnp.float32)


def _proj_body(hv, w1a, b1, w1c, pre1_o, g1_o):
    x = hv[...]
    pre1_o[...] = (
        jnp.dot(x, w1a[...], preferred_element_type=jnp.float32) + b1[...]
    )
    g1_o[...] = jnp.dot(x, w1c[...], preferred_element_type=jnp.float32)


def _block1_body(hv, pre1, he, g1, w1b, w2, b2, w3, b3, win, bi, wout, bo,
                 n1g, n1b, n2g, n2b, w11a, b11, w11c,
                 hv2_o, pre11_o, g11_o):
    nb, Hd = hv.shape
    ne = he.shape[0]
    K = ne // nb
    e = _bdot(he[...], w1b[...])
    t = (e + g1[...].astype(jnp.float32)).reshape(nb, K, Hd) + pre1[...][:, None, :]
    t = _gelu(t).reshape(ne, Hd)
    u = _gelu(_bdot(t, w2[...]) + b2[...])
    msg = _bdot(u, w3[...]) + b3[...]
    dh = msg.reshape(nb, K, Hd).sum(axis=1) * (1.0 / 30.0)
    h = _ln(hv[...] + dh, n1g[...], n1b[...])
    f = _gelu(jnp.dot(h, win[...], preferred_element_type=jnp.float32) + bi[...])
    f = jnp.dot(f, wout[...], preferred_element_type=jnp.float32) + bo[...]
    y = _ln(h + f, n2g[...], n2b[...])
    hv2_o[...] = y
    pre11_o[...] = (
        jnp.dot(y, w11a[...], preferred_element_type=jnp.float32) + b11[...]
    )
    g11_o[...] = jnp.dot(y, w11c[...], preferred_element_type=jnp.float32)


def _block2_body(he, g2, pre11, w11b, w12, b12, w13, b13, n3g, n3b, heo):
    ne, Hd = he.shape
    nb = pre11.shape[0]
    K = ne // nb
    e = _bdot(he[...], w11b[...])
    t = (e + g2[...].astype(jnp.float32)).reshape(nb, K, Hd) + pre11[...][:, None, :]
    t = _gelu(t).reshape(ne, Hd)
    u = _gelu(_bdot(t, w12[...]) + b12[...])
    msg = _bdot(u, w13[...]) + b13[...]
    heo[...] = _ln(he[...] + msg, n3g[...], n3b[...])


def _row_spec(rows, cols, off=0):
    return pl.BlockSpec((rows, cols), lambda i: (i + off, 0))


def _const_spec(rows, cols):
    return pl.BlockSpec((rows, cols), lambda i: (0, 0))


def _block1_alias(hv, pre1, he, g1, w1b, w2, b2, w3, b3, win, bi, wout, bo,
                  n1g, n1b, n2g, n2b, w11a, b11, w11c, d0, d1, d2,
                  hv2_o, pre11_o, g11_o):
    del d0, d1, d2
    _block1_body(hv, pre1, he, g1, w1b, w2, b2, w3, b3, win, bi, wout, bo,
                 n1g, n1b, n2g, n2b, w11a, b11, w11c,
                 hv2_o, pre11_o, g11_o)


def _block2_alias(he, g2, pre11, w11b, w12, b12, w13, b13, n3g, n3b, d0, heo):
    del d0
    _block2_body(he, g2, pre11, w11b, w12, b12, w13, b13, n3g, n3b, heo)


def _pad_idx(idx, align):
    m = idx.shape[0]
    mp = ((m + align - 1) // align) * align
    if mp != m:
        idx = jnp.concatenate([idx, jnp.zeros((mp - m,), jnp.int32)])
    return idx


def kernel(h_V, h_E, E_idx, params):
    B, N, Hd = h_V.shape
    K = h_E.shape[2]
    M = N * K
    hv = h_V.reshape(N, Hd)
    he = h_E.reshape(M, Hd)
    idx = E_idx.reshape(-1).astype(jnp.int32)

    p = params
    w1 = p["W1"]["w"]
    w11 = p["W11"]["w"]
    w1a, w1b, w1c = w1[:Hd], w1[Hd:2 * Hd], w1[2 * Hd:]
    w11a, w11b, w11c = w11[:Hd], w11[Hd:2 * Hd], w11[2 * Hd:]
    b1 = p["W1"]["b"].reshape(1, Hd)
    b11 = p["W11"]["b"].reshape(1, Hd)
    b2 = p["W2"]["b"].reshape(1, Hd)
    b3 = p["W3"]["b"].reshape(1, Hd)
    b12 = p["W12"]["b"].reshape(1, Hd)
    b13 = p["W13"]["b"].reshape(1, Hd)
    bi = p["Win"]["b"].reshape(1, -1)
    bo = p["Wout"]["b"].reshape(1, Hd)
    n1g = p["norm1"]["g"].reshape(1, Hd)
    n1b = p["norm1"]["b"].reshape(1, Hd)
    n2g = p["norm2"]["g"].reshape(1, Hd)
    n2b = p["norm2"]["b"].reshape(1, Hd)
    n3g = p["norm3"]["g"].reshape(1, Hd)
    n3b = p["norm3"]["b"].reshape(1, Hd)
    Hi = p["Win"]["w"].shape[1]

    align = _GC * _WORKERS
    ne = _NB * K

    S = 7600
    nb1 = S // _NB
    nb2 = (N - S) // _NB
    M1 = S * K
    idx1 = _pad_idx(idx[:M1], align)
    idx2 = _pad_idx(idx[M1:], align)

    pre1, g1t = pl.pallas_call(
        _proj_body,
        out_shape=[
            jax.ShapeDtypeStruct((N, Hd), jnp.float32),
            jax.ShapeDtypeStruct((N, Hd), jnp.float32),
        ],
    )(hv, w1a, b1, w1c)

    g1a = _sc_gather(g1t, idx1)
    g1b = _sc_gather(g1t, idx2)

    b_consts = [
        _const_spec(Hd, Hd),
        _const_spec(Hd, Hd),
        _const_spec(1, Hd),
        _const_spec(Hd, Hd),
        _const_spec(1, Hd),
        _const_spec(Hd, Hi),
        _const_spec(1, Hi),
        _const_spec(Hi, Hd),
        _const_spec(1, Hd),
        _const_spec(1, Hd),
        _const_spec(1, Hd),
        _const_spec(1, Hd),
        _const_spec(1, Hd),
        _const_spec(Hd, Hd),
        _const_spec(1, Hd),
        _const_spec(Hd, Hd),
    ]
    b_args = (w1b, p["W2"]["w"], b2, p["W3"]["w"], b3,
              p["Win"]["w"], bi, p["Wout"]["w"], bo, n1g, n1b, n2g, n2b,
              w11a, b11, w11c)
    b_out_shape = [
        jax.ShapeDtypeStruct((N, Hd), jnp.float32),
        jax.ShapeDtypeStruct((N, Hd), jnp.float32),
        jax.ShapeDtypeStruct((N, Hd), jnp.float32),
    ]

    hv2a, pre11a, g11a = pl.pallas_call(
        _block1_body,
        grid=(nb1,),
        in_specs=[
            _row_spec(_NB, Hd),
            _row_spec(_NB, Hd),
            _row_spec(ne, Hd),
            _row_spec(ne, Hd),
        ] + b_consts,
        out_specs=[
            _row_spec(_NB, Hd),
            _row_spec(_NB, Hd),
            _row_spec(_NB, Hd),
        ],
        out_shape=b_out_shape,
    )(hv, pre1, he, g1a, *b_args)

    hv2, pre11, g11t = pl.pallas_call(
        _block1_alias,
        grid=(nb2,),
        in_specs=[
            _row_spec(_NB, Hd, nb1),
            _row_spec(_NB, Hd, nb1),
            _row_spec(ne, Hd, nb1),
            _row_spec(ne, Hd),
        ] + b_consts + [
            pl.BlockSpec((8, Hd), lambda i: (0, 0)),
            pl.BlockSpec((8, Hd), lambda i: (0, 0)),
            pl.BlockSpec((8, Hd), lambda i: (0, 0)),
        ],
        out_specs=[
            _row_spec(_NB, Hd, nb1),
            _row_spec(_NB, Hd, nb1),
            _row_spec(_NB, Hd, nb1),
        ],
        out_shape=b_out_shape,
        input_output_aliases={20: 0, 21: 1, 22: 2},
    )(hv, pre1, he, g1b, *b_args, hv2a, pre11a, g11a)

    g2a = _sc_gather(g11t, idx1)
    g2b = _sc_gather(g11t, idx2)

    c_consts = [
        _const_spec(Hd, Hd),
        _const_spec(Hd, Hd),
        _const_spec(1, Hd),
        _const_spec(Hd, Hd),
        _const_spec(1, Hd),
        _const_spec(1, Hd),
        _const_spec(1, Hd),
    ]
    c_args = (w11b, p["W12"]["w"], b12, p["W13"]["w"], b13, n3g, n3b)

    heo1 = pl.pallas_call(
        _block2_body,
        grid=(nb1,),
        in_specs=[
            _row_spec(ne, Hd),
            _row_spec(ne, Hd),
            _row_spec(_NB, Hd),
        ] + c_consts,
        out_specs=[_row_spec(ne, Hd)],
        out_shape=[jax.ShapeDtypeStruct((M, Hd), jnp.float32)],
    )(he, g2a, pre11, *c_args)[0]

    heo = pl.pallas_call(
        _block2_alias,
        grid=(nb2,),
        in_specs=[
            _row_spec(ne, Hd, nb1),
            _row_spec(ne, Hd),
            _row_spec(_NB, Hd, nb1),
        ] + c_consts + [
            pl.BlockSpec((8, Hd), lambda i: (0, 0)),
        ],
        out_specs=[_row_spec(ne, Hd, nb1)],
        out_shape=[jax.ShapeDtypeStruct((M, Hd), jnp.float32)],
        input_output_aliases={10: 0},
    )(he, g2b, pre11, *c_args, heo1)[0]

    return hv2.reshape(B, N, Hd), heo.reshape(B, N, K, Hd)


# speedup vs baseline: 1.6105x; 1.0304x over previous
import functools

import jax
import jax.numpy as jnp
from jax.experimental import pallas as pl
from jax.experimental.pallas import tpu as pltpu
from jax.experimental.pallas import tpu_sc as plsc

_GC = 128
_WORKERS = 32
_NB = 200


def _gelu(x):
    return x * 0.5 * (1.0 + jax.lax.erf(x * 0.7071067811865476))


def _ln(x, g, b):
    m = jnp.mean(x, axis=-1, keepdims=True)
    d = x - m
    v = jnp.mean(d * d, axis=-1, keepdims=True)
    return d * jax.lax.rsqrt(v + 1e-5) * g + b


def _sc_gather(table, idx1d):
    Mp = idx1d.shape[0]
    D = table.shape[1]
    idx2d = idx1d.reshape(1, Mp)
    mesh = plsc.VectorSubcoreMesh(core_axis_name="c", subcore_axis_name="s")

    @functools.partial(
        pl.kernel,
        out_type=jax.ShapeDtypeStruct((Mp, D), table.dtype),
        mesh=mesh,
    )
    def run(x_hbm, i_hbm, o_hbm):
        def body(i_vmem, o_vmem):
            pltpu.sync_copy(x_hbm.at[i_vmem.at[0]], o_vmem)

        pltpu.emit_pipeline(
            body,
            grid=(Mp // _GC,),
            in_specs=[pl.BlockSpec((1, _GC), lambda i: (0, i))],
            out_specs=[pl.BlockSpec((_GC, D), lambda i: (i, 0))],
            core_axis_name=("c", "s"),
            dimension_semantics=(pltpu.PARALLEL,),
        )(i_hbm, o_hbm)

    return run(table, idx2d)


def _bdot(a, b):
    return jnp.dot(a.astype(jnp.bfloat16), b.astype(jnp.bfloat16),
                   preferred_element_type=jnp.float32)


def _proj_body(hv, w1a, b1, w1c, pre1_o, g1_o):
    x = hv[...]
    pre1_o[...] = (
        jnp.dot(x, w1a[...], preferred_element_type=jnp.float32) + b1[...]
    )
    g1_o[...] = jnp.dot(x, w1c[...], preferred_element_type=jnp.float32)


def _block1_body(hv, pre1, he, g1, w1b, w2, b2, w3, b3, win, bi, wout, bo,
                 n1g, n1b, n2g, n2b, w11a, b11, w11c,
                 hv2_o, pre11_o, g11_o):
    nb, Hd = hv.shape
    ne = he.shape[0]
    K = ne // nb
    e = _bdot(he[...], w1b[...])
    t = (e + g1[...].astype(jnp.float32)).reshape(nb, K, Hd) + pre1[...][:, None, :]
    t = _gelu(t).reshape(ne, Hd)
    u = _gelu(_bdot(t, w2[...]) + b2[...])
    msg = _bdot(u, w3[...]) + b3[...]
    dh = msg.reshape(nb, K, Hd).sum(axis=1) * (1.0 / 30.0)
    h = _ln(hv[...] + dh, n1g[...], n1b[...])
    f = _gelu(jnp.dot(h, win[...], preferred_element_type=jnp.float32) + bi[...])
    f = jnp.dot(f, wout[...], preferred_element_type=jnp.float32) + bo[...]
    y = _ln(h + f, n2g[...], n2b[...])
    hv2_o[...] = y
    pre11_o[...] = (
        jnp.dot(y, w11a[...], preferred_element_type=jnp.float32) + b11[...]
    )
    g11_o[...] = jnp.dot(y, w11c[...], preferred_element_type=jnp.float32)


def _block2_body(he, g2, pre11, w11b, w12, b12, w13, b13, n3g, n3b, heo):
    ne, Hd = he.shape
    nb = pre11.shape[0]
    K = ne // nb
    e = _bdot(he[...], w11b[...])
    t = (e + g2[...].astype(jnp.float32)).reshape(nb, K, Hd) + pre11[...][:, None, :]
    t = _gelu(t).reshape(ne, Hd)
    u = _gelu(_bdot(t, w12[...]) + b12[...])
    msg = _bdot(u, w13[...]) + b13[...]
    heo[...] = _ln(he[...] + msg, n3g[...], n3b[...])


def _row_spec(rows, cols, off=0):
    return pl.BlockSpec((rows, cols), lambda i: (i + off, 0))


def _const_spec(rows, cols):
    return pl.BlockSpec((rows, cols), lambda i: (0, 0))


def _block1_alias(hv, pre1, he, g1, w1b, w2, b2, w3, b3, win, bi, wout, bo,
                  n1g, n1b, n2g, n2b, w11a, b11, w11c, d0, d1, d2,
                  hv2_o, pre11_o, g11_o):
    del d0, d1, d2
    _block1_body(hv, pre1, he, g1, w1b, w2, b2, w3, b3, win, bi, wout, bo,
                 n1g, n1b, n2g, n2b, w11a, b11, w11c,
                 hv2_o, pre11_o, g11_o)


def _block2_alias(he, g2, pre11, w11b, w12, b12, w13, b13, n3g, n3b, d0, heo):
    del d0
    _block2_body(he, g2, pre11, w11b, w12, b12, w13, b13, n3g, n3b, heo)


def _pad_idx(idx, align):
    m = idx.shape[0]
    mp = ((m + align - 1) // align) * align
    if mp != m:
        idx = jnp.concatenate([idx, jnp.zeros((mp - m,), jnp.int32)])
    return idx


def kernel(h_V, h_E, E_idx, params):
    B, N, Hd = h_V.shape
    K = h_E.shape[2]
    M = N * K
    hv = h_V.reshape(N, Hd)
    he = h_E.reshape(M, Hd)
    idx = E_idx.reshape(-1).astype(jnp.int32)

    p = params
    w1 = p["W1"]["w"]
    w11 = p["W11"]["w"]
    w1a, w1b, w1c = w1[:Hd], w1[Hd:2 * Hd], w1[2 * Hd:]
    w11a, w11b, w11c = w11[:Hd], w11[Hd:2 * Hd], w11[2 * Hd:]
    b1 = p["W1"]["b"].reshape(1, Hd)
    b11 = p["W11"]["b"].reshape(1, Hd)
    b2 = p["W2"]["b"].reshape(1, Hd)
    b3 = p["W3"]["b"].reshape(1, Hd)
    b12 = p["W12"]["b"].reshape(1, Hd)
    b13 = p["W13"]["b"].reshape(1, Hd)
    bi = p["Win"]["b"].reshape(1, -1)
    bo = p["Wout"]["b"].reshape(1, Hd)
    n1g = p["norm1"]["g"].reshape(1, Hd)
    n1b = p["norm1"]["b"].reshape(1, Hd)
    n2g = p["norm2"]["g"].reshape(1, Hd)
    n2b = p["norm2"]["b"].reshape(1, Hd)
    n3g = p["norm3"]["g"].reshape(1, Hd)
    n3b = p["norm3"]["b"].reshape(1, Hd)
    Hi = p["Win"]["w"].shape[1]

    align = _GC * _WORKERS
    ne = _NB * K

    S = 8000
    nb1 = S // _NB
    nb2 = (N - S) // _NB
    M1 = S * K
    idx1 = _pad_idx(idx[:M1], align)
    idx2 = _pad_idx(idx[M1:], align)

    pre1, g1t = pl.pallas_call(
        _proj_body,
        out_shape=[
            jax.ShapeDtypeStruct((N, Hd), jnp.float32),
            jax.ShapeDtypeStruct((N, Hd), jnp.float32),
        ],
    )(hv, w1a, b1, w1c)

    g1a = _sc_gather(g1t, idx1)
    g1b = _sc_gather(g1t, idx2)

    b_consts = [
        _const_spec(Hd, Hd),
        _const_spec(Hd, Hd),
        _const_spec(1, Hd),
        _const_spec(Hd, Hd),
        _const_spec(1, Hd),
        _const_spec(Hd, Hi),
        _const_spec(1, Hi),
        _const_spec(Hi, Hd),
        _const_spec(1, Hd),
        _const_spec(1, Hd),
        _const_spec(1, Hd),
        _const_spec(1, Hd),
        _const_spec(1, Hd),
        _const_spec(Hd, Hd),
        _const_spec(1, Hd),
        _const_spec(Hd, Hd),
    ]
    b_args = (w1b, p["W2"]["w"], b2, p["W3"]["w"], b3,
              p["Win"]["w"], bi, p["Wout"]["w"], bo, n1g, n1b, n2g, n2b,
              w11a, b11, w11c)
    b_out_shape = [
        jax.ShapeDtypeStruct((N, Hd), jnp.float32),
        jax.ShapeDtypeStruct((N, Hd), jnp.float32),
        jax.ShapeDtypeStruct((N, Hd), jnp.float32),
    ]

    hv2a, pre11a, g11a = pl.pallas_call(
        _block1_body,
        grid=(nb1,),
        in_specs=[
            _row_spec(_NB, Hd),
            _row_spec(_NB, Hd),
            _row_spec(ne, Hd),
            _row_spec(ne, Hd),
        ] + b_consts,
        out_specs=[
            _row_spec(_NB, Hd),
            _row_spec(_NB, Hd),
            _row_spec(_NB, Hd),
        ],
        out_shape=b_out_shape,
    )(hv, pre1, he, g1a, *b_args)

    hv2, pre11, g11t = pl.pallas_call(
        _block1_alias,
        grid=(nb2,),
        in_specs=[
            _row_spec(_NB, Hd, nb1),
            _row_spec(_NB, Hd, nb1),
            _row_spec(ne, Hd, nb1),
            _row_spec(ne, Hd),
        ] + b_consts + [
            pl.BlockSpec((8, Hd), lambda i: (0, 0)),
            pl.BlockSpec((8, Hd), lambda i: (0, 0)),
            pl.BlockSpec((8, Hd), lambda i: (0, 0)),
        ],
        out_specs=[
            _row_spec(_NB, Hd, nb1),
            _row_spec(_NB, Hd, nb1),
            _row_spec(_NB, Hd, nb1),
        ],
        out_shape=b_out_shape,
        input_output_aliases={20: 0, 21: 1, 22: 2},
    )(hv, pre1, he, g1b, *b_args, hv2a, pre11a, g11a)

    g2a = _sc_gather(g11t, idx1)
    g2b = _sc_gather(g11t, idx2)

    c_consts = [
        _const_spec(Hd, Hd),
        _const_spec(Hd, Hd),
        _const_spec(1, Hd),
        _const_spec(Hd, Hd),
        _const_spec(1, Hd),
        _const_spec(1, Hd),
        _const_spec(1, Hd),
    ]
    c_args = (w11b, p["W12"]["w"], b12, p["W13"]["w"], b13, n3g, n3b)

    heo1 = pl.pallas_call(
        _block2_body,
        grid=(nb1,),
        in_specs=[
            _row_spec(ne, Hd),
            _row_spec(ne, Hd),
            _row_spec(_NB, Hd),
        ] + c_consts,
        out_specs=[_row_spec(ne, Hd)],
        out_shape=[jax.ShapeDtypeStruct((M, Hd), jnp.float32)],
    )(he, g2a, pre11, *c_args)[0]

    heo = pl.pallas_call(
        _block2_alias,
        grid=(nb2,),
        in_specs=[
            _row_spec(ne, Hd, nb1),
            _row_spec(ne, Hd),
            _row_spec(_NB, Hd, nb1),
        ] + c_consts + [
            pl.BlockSpec((8, Hd), lambda i: (0, 0)),
        ],
        out_specs=[_row_spec(ne, Hd, nb1)],
        out_shape=[jax.ShapeDtypeStruct((M, Hd), jnp.float32)],
        input_output_aliases={10: 0},
    )(he, g2b, pre11, *c_args, heo1)[0]

    return hv2.reshape(B, N, Hd), heo.reshape(B, N, K, Hd)


# speedup vs baseline: 1.6409x; 1.0188x over previous
import functools

import jax
import jax.numpy as jnp
from jax.experimental import pallas as pl
from jax.experimental.pallas import tpu as pltpu
from jax.experimental.pallas import tpu_sc as plsc

_GC = 128
_WORKERS = 32
_NB = 200


def _gelu(x):
    return x * 0.5 * (1.0 + jax.lax.erf(x * 0.7071067811865476))


def _ln(x, g, b):
    m = jnp.mean(x, axis=-1, keepdims=True)
    d = x - m
    v = jnp.mean(d * d, axis=-1, keepdims=True)
    return d * jax.lax.rsqrt(v + 1e-5) * g + b


def _sc_gather(table, idx1d):
    Mp = idx1d.shape[0]
    D = table.shape[1]
    idx2d = idx1d.reshape(1, Mp)
    mesh = plsc.VectorSubcoreMesh(core_axis_name="c", subcore_axis_name="s")

    @functools.partial(
        pl.kernel,
        out_type=jax.ShapeDtypeStruct((Mp, D), table.dtype),
        mesh=mesh,
    )
    def run(x_hbm, i_hbm, o_hbm):
        def body(i_vmem, o_vmem):
            pltpu.sync_copy(x_hbm.at[i_vmem.at[0]], o_vmem)

        pltpu.emit_pipeline(
            body,
            grid=(Mp // _GC,),
            in_specs=[pl.BlockSpec((1, _GC), lambda i: (0, i))],
            out_specs=[pl.BlockSpec((_GC, D), lambda i: (i, 0))],
            core_axis_name=("c", "s"),
            dimension_semantics=(pltpu.PARALLEL,),
        )(i_hbm, o_hbm)

    return run(table, idx2d)


def _bdot(a, b):
    return jnp.dot(a.astype(jnp.bfloat16), b.astype(jnp.bfloat16),
                   preferred_element_type=jnp.float32)


def _proj_body(hv, w1a, b1, w1c, pre1_o, g1_o):
    x = hv[...]
    pre1_o[...] = (
        jnp.dot(x, w1a[...], preferred_element_type=jnp.float32) + b1[...]
    )
    g1_o[...] = jnp.dot(x, w1c[...], preferred_element_type=jnp.float32)


def _block1_body(hv, pre1, he, g1, w1b, w2, b2, w3, b3, win, bi, wout, bo,
                 n1g, n1b, n2g, n2b, w11a, b11, w11c,
                 hv2_o, pre11_o, g11_o):
    nb, Hd = hv.shape
    ne = he.shape[0]
    K = ne // nb
    e = _bdot(he[...], w1b[...])
    t = (e + g1[...].astype(jnp.float32)).reshape(nb, K, Hd) + pre1[...][:, None, :]
    t = _gelu(t).reshape(ne, Hd)
    u = _gelu(_bdot(t, w2[...]) + b2[...])
    msg = _bdot(u, w3[...]) + b3[...]
    dh = msg.reshape(nb, K, Hd).sum(axis=1) * (1.0 / 30.0)
    h = _ln(hv[...] + dh, n1g[...], n1b[...])
    f = _gelu(jnp.dot(h, win[...], preferred_element_type=jnp.float32) + bi[...])
    f = jnp.dot(f, wout[...], preferred_element_type=jnp.float32) + bo[...]
    y = _ln(h + f, n2g[...], n2b[...])
    hv2_o[...] = y
    pre11_o[...] = (
        jnp.dot(y, w11a[...], preferred_element_type=jnp.float32) + b11[...]
    )
    g11_o[...] = jnp.dot(y, w11c[...], preferred_element_type=jnp.float32)


def _block2_body(he, g2, pre11, w11b, w12, b12, w13, b13, n3g, n3b, heo):
    ne, Hd = he.shape
    nb = pre11.shape[0]
    K = ne // nb
    e = _bdot(he[...], w11b[...])
    t = (e + g2[...].astype(jnp.float32)).reshape(nb, K, Hd) + pre11[...][:, None, :]
    t = _gelu(t).reshape(ne, Hd)
    u = _gelu(_bdot(t, w12[...]) + b12[...])
    msg = _bdot(u, w13[...]) + b13[...]
    heo[...] = _ln(he[...] + msg, n3g[...], n3b[...])


def _row_spec(rows, cols, off=0):
    return pl.BlockSpec((rows, cols), lambda i: (i + off, 0))


def _const_spec(rows, cols):
    return pl.BlockSpec((rows, cols), lambda i: (0, 0))


def _block1_alias(hv, pre1, he, g1, w1b, w2, b2, w3, b3, win, bi, wout, bo,
                  n1g, n1b, n2g, n2b, w11a, b11, w11c, d0, d1, d2,
                  hv2_o, pre11_o, g11_o):
    del d0, d1, d2
    _block1_body(hv, pre1, he, g1, w1b, w2, b2, w3, b3, win, bi, wout, bo,
                 n1g, n1b, n2g, n2b, w11a, b11, w11c,
                 hv2_o, pre11_o, g11_o)


def _block2_alias(he, g2, pre11, w11b, w12, b12, w13, b13, n3g, n3b, d0, heo):
    del d0
    _block2_body(he, g2, pre11, w11b, w12, b12, w13, b13, n3g, n3b, heo)


def _pad_idx(idx, align):
    m = idx.shape[0]
    mp = ((m + align - 1) // align) * align
    if mp != m:
        idx = jnp.concatenate([idx, jnp.zeros((mp - m,), jnp.int32)])
    return idx


def kernel(h_V, h_E, E_idx, params):
    B, N, Hd = h_V.shape
    K = h_E.shape[2]
    M = N * K
    hv = h_V.reshape(N, Hd)
    he = h_E.reshape(M, Hd)
    idx = E_idx.reshape(-1).astype(jnp.int32)

    p = params
    w1 = p["W1"]["w"]
    w11 = p["W11"]["w"]
    w1a, w1b, w1c = w1[:Hd], w1[Hd:2 * Hd], w1[2 * Hd:]
    w11a, w11b, w11c = w11[:Hd], w11[Hd:2 * Hd], w11[2 * Hd:]
    b1 = p["W1"]["b"].reshape(1, Hd)
    b11 = p["W11"]["b"].reshape(1, Hd)
    b2 = p["W2"]["b"].reshape(1, Hd)
    b3 = p["W3"]["b"].reshape(1, Hd)
    b12 = p["W12"]["b"].reshape(1, Hd)
    b13 = p["W13"]["b"].reshape(1, Hd)
    bi = p["Win"]["b"].reshape(1, -1)
    bo = p["Wout"]["b"].reshape(1, Hd)
    n1g = p["norm1"]["g"].reshape(1, Hd)
    n1b = p["norm1"]["b"].reshape(1, Hd)
    n2g = p["norm2"]["g"].reshape(1, Hd)
    n2b = p["norm2"]["b"].reshape(1, Hd)
    n3g = p["norm3"]["g"].reshape(1, Hd)
    n3b = p["norm3"]["b"].reshape(1, Hd)
    Hi = p["Win"]["w"].shape[1]

    align = _GC * _WORKERS
    ne = _NB * K

    S = 8400
    nb1 = S // _NB
    nb2 = (N - S) // _NB
    M1 = S * K
    idx1 = _pad_idx(idx[:M1], align)
    idx2 = _pad_idx(idx[M1:], align)

    pre1, g1t = pl.pallas_call(
        _proj_body,
        out_shape=[
            jax.ShapeDtypeStruct((N, Hd), jnp.float32),
            jax.ShapeDtypeStruct((N, Hd), jnp.float32),
        ],
    )(hv, w1a, b1, w1c)

    g1a = _sc_gather(g1t, idx1)
    g1b = _sc_gather(g1t, idx2)

    b_consts = [
        _const_spec(Hd, Hd),
        _const_spec(Hd, Hd),
        _const_spec(1, Hd),
        _const_spec(Hd, Hd),
        _const_spec(1, Hd),
        _const_spec(Hd, Hi),
        _const_spec(1, Hi),
        _const_spec(Hi, Hd),
        _const_spec(1, Hd),
        _const_spec(1, Hd),
        _const_spec(1, Hd),
        _const_spec(1, Hd),
        _const_spec(1, Hd),
        _const_spec(Hd, Hd),
        _const_spec(1, Hd),
        _const_spec(Hd, Hd),
    ]
    b_args = (w1b, p["W2"]["w"], b2, p["W3"]["w"], b3,
              p["Win"]["w"], bi, p["Wout"]["w"], bo, n1g, n1b, n2g, n2b,
              w11a, b11, w11c)
    b_out_shape = [
        jax.ShapeDtypeStruct((N, Hd), jnp.float32),
        jax.ShapeDtypeStruct((N, Hd), jnp.float32),
        jax.ShapeDtypeStruct((N, Hd), jnp.float32),
    ]

    hv2a, pre11a, g11a = pl.pallas_call(
        _block1_body,
        grid=(nb1,),
        in_specs=[
            _row_spec(_NB, Hd),
            _row_spec(_NB, Hd),
            _row_spec(ne, Hd),
            _row_spec(ne, Hd),
        ] + b_consts,
        out_specs=[
            _row_spec(_NB, Hd),
            _row_spec(_NB, Hd),
            _row_spec(_NB, Hd),
        ],
        out_shape=b_out_shape,
    )(hv, pre1, he, g1a, *b_args)

    hv2, pre11, g11t = pl.pallas_call(
        _block1_alias,
        grid=(nb2,),
        in_specs=[
            _row_spec(_NB, Hd, nb1),
            _row_spec(_NB, Hd, nb1),
            _row_spec(ne, Hd, nb1),
            _row_spec(ne, Hd),
        ] + b_consts + [
            pl.BlockSpec((8, Hd), lambda i: (0, 0)),
            pl.BlockSpec((8, Hd), lambda i: (0, 0)),
            pl.BlockSpec((8, Hd), lambda i: (0, 0)),
        ],
        out_specs=[
            _row_spec(_NB, Hd, nb1),
            _row_spec(_NB, Hd, nb1),
            _row_spec(_NB, Hd, nb1),
        ],
        out_shape=b_out_shape,
        input_output_aliases={20: 0, 21: 1, 22: 2},
    )(hv, pre1, he, g1b, *b_args, hv2a, pre11a, g11a)

    g2a = _sc_gather(g11t, idx1)
    g2b = _sc_gather(g11t, idx2)

    c_consts = [
        _const_spec(Hd, Hd),
        _const_spec(Hd, Hd),
        _const_spec(1, Hd),
        _const_spec(Hd, Hd),
        _const_spec(1, Hd),
        _const_spec(1, Hd),
        _const_spec(1, Hd),
    ]
    c_args = (w11b, p["W12"]["w"], b12, p["W13"]["w"], b13, n3g, n3b)

    heo1 = pl.pallas_call(
        _block2_body,
        grid=(nb1,),
        in_specs=[
            _row_spec(ne, Hd),
            _row_spec(ne, Hd),
            _row_spec(_NB, Hd),
        ] + c_consts,
        out_specs=[_row_spec(ne, Hd)],
        out_shape=[jax.ShapeDtypeStruct((M, Hd), jnp.float32)],
    )(he, g2a, pre11, *c_args)[0]

    heo = pl.pallas_call(
        _block2_alias,
        grid=(nb2,),
        in_specs=[
            _row_spec(ne, Hd, nb1),
            _row_spec(ne, Hd),
            _row_spec(_NB, Hd, nb1),
        ] + c_consts + [
            pl.BlockSpec((8, Hd), lambda i: (0, 0)),
        ],
        out_specs=[_row_spec(ne, Hd, nb1)],
        out_shape=[jax.ShapeDtypeStruct((M, Hd), jnp.float32)],
        input_output_aliases={10: 0},
    )(he, g2b, pre11, *c_args, heo1)[0]

    return hv2.reshape(B, N, Hd), heo.reshape(B, N, K, Hd)


# speedup vs baseline: 1.6424x; 1.0009x over previous
import functools

import jax
import jax.numpy as jnp
from jax.experimental import pallas as pl
from jax.experimental.pallas import tpu as pltpu
from jax.experimental.pallas import tpu_sc as plsc

_GC = 128
_WORKERS = 32
_NB = 200


def _gelu(x):
    return x * 0.5 * (1.0 + jax.lax.erf(x * 0.7071067811865476))


def _ln(x, g, b):
    m = jnp.mean(x, axis=-1, keepdims=True)
    d = x - m
    v = jnp.mean(d * d, axis=-1, keepdims=True)
    return d * jax.lax.rsqrt(v + 1e-5) * g + b


def _sc_gather(table, idx1d):
    Mp = idx1d.shape[0]
    D = table.shape[1]
    idx2d = idx1d.reshape(1, Mp)
    mesh = plsc.VectorSubcoreMesh(core_axis_name="c", subcore_axis_name="s")

    @functools.partial(
        pl.kernel,
        out_type=jax.ShapeDtypeStruct((Mp, D), table.dtype),
        mesh=mesh,
    )
    def run(x_hbm, i_hbm, o_hbm):
        def body(i_vmem, o_vmem):
            pltpu.sync_copy(x_hbm.at[i_vmem.at[0]], o_vmem)

        pltpu.emit_pipeline(
            body,
            grid=(Mp // _GC,),
            in_specs=[pl.BlockSpec((1, _GC), lambda i: (0, i))],
            out_specs=[pl.BlockSpec((_GC, D), lambda i: (i, 0))],
            core_axis_name=("c", "s"),
            dimension_semantics=(pltpu.PARALLEL,),
        )(i_hbm, o_hbm)

    return run(table, idx2d)


def _bdot(a, b):
    return jnp.dot(a.astype(jnp.bfloat16), b.astype(jnp.bfloat16),
                   preferred_element_type=jnp.float32)


def _proj_body(hv, w1a, b1, w1c, pre1_o, g1_o):
    x = hv[...]
    pre1_o[...] = (
        jnp.dot(x, w1a[...], preferred_element_type=jnp.float32) + b1[...]
    )
    g1_o[...] = jnp.dot(x, w1c[...], preferred_element_type=jnp.float32)


def _block1_body(hv, pre1, he, g1, w1b, w2, b2, w3, b3, win, bi, wout, bo,
                 n1g, n1b, n2g, n2b, w11a, b11, w11c,
                 hv2_o, pre11_o, g11_o):
    nb, Hd = hv.shape
    ne = he.shape[0]
    K = ne // nb
    e = _bdot(he[...], w1b[...])
    t = (e + g1[...].astype(jnp.float32)).reshape(nb, K, Hd) + pre1[...][:, None, :]
    t = _gelu(t).reshape(ne, Hd)
    u = _gelu(_bdot(t, w2[...]) + b2[...])
    msg = _bdot(u, w3[...]) + b3[...]
    dh = msg.reshape(nb, K, Hd).sum(axis=1) * (1.0 / 30.0)
    h = _ln(hv[...] + dh, n1g[...], n1b[...])
    f = _gelu(jnp.dot(h, win[...], preferred_element_type=jnp.float32) + bi[...])
    f = jnp.dot(f, wout[...], preferred_element_type=jnp.float32) + bo[...]
    y = _ln(h + f, n2g[...], n2b[...])
    hv2_o[...] = y
    pre11_o[...] = (
        jnp.dot(y, w11a[...], preferred_element_type=jnp.float32) + b11[...]
    )
    g11_o[...] = jnp.dot(y, w11c[...], preferred_element_type=jnp.float32)


def _block2_body(he, g2, pre11, w11b, w12, b12, w13, b13, n3g, n3b, heo):
    ne, Hd = he.shape
    nb = pre11.shape[0]
    K = ne // nb
    e = _bdot(he[...], w11b[...])
    t = (e + g2[...].astype(jnp.float32)).reshape(nb, K, Hd) + pre11[...][:, None, :]
    t = _gelu(t).reshape(ne, Hd)
    u = _gelu(_bdot(t, w12[...]) + b12[...])
    msg = _bdot(u, w13[...]) + b13[...]
    heo[...] = _ln(he[...] + msg, n3g[...], n3b[...])


def _row_spec(rows, cols, off=0):
    return pl.BlockSpec((rows, cols), lambda i: (i + off, 0))


def _const_spec(rows, cols):
    return pl.BlockSpec((rows, cols), lambda i: (0, 0))


def _block1_alias(hv, pre1, he, g1, w1b, w2, b2, w3, b3, win, bi, wout, bo,
                  n1g, n1b, n2g, n2b, w11a, b11, w11c, d0, d1, d2,
                  hv2_o, pre11_o, g11_o):
    del d0, d1, d2
    _block1_body(hv, pre1, he, g1, w1b, w2, b2, w3, b3, win, bi, wout, bo,
                 n1g, n1b, n2g, n2b, w11a, b11, w11c,
                 hv2_o, pre11_o, g11_o)


def _block2_alias(he, g2, pre11, w11b, w12, b12, w13, b13, n3g, n3b, d0, heo):
    del d0
    _block2_body(he, g2, pre11, w11b, w12, b12, w13, b13, n3g, n3b, heo)


def _pad_idx(idx, align):
    m = idx.shape[0]
    mp = ((m + align - 1) // align) * align
    if mp != m:
        idx = jnp.concatenate([idx, jnp.zeros((mp - m,), jnp.int32)])
    return idx


def kernel(h_V, h_E, E_idx, params):
    B, N, Hd = h_V.shape
    K = h_E.shape[2]
    M = N * K
    hv = h_V.reshape(N, Hd)
    he = h_E.reshape(M, Hd)
    idx = E_idx.reshape(-1).astype(jnp.int32)

    p = params
    w1 = p["W1"]["w"]
    w11 = p["W11"]["w"]
    w1a, w1b, w1c = w1[:Hd], w1[Hd:2 * Hd], w1[2 * Hd:]
    w11a, w11b, w11c = w11[:Hd], w11[Hd:2 * Hd], w11[2 * Hd:]
    b1 = p["W1"]["b"].reshape(1, Hd)
    b11 = p["W11"]["b"].reshape(1, Hd)
    b2 = p["W2"]["b"].reshape(1, Hd)
    b3 = p["W3"]["b"].reshape(1, Hd)
    b12 = p["W12"]["b"].reshape(1, Hd)
    b13 = p["W13"]["b"].reshape(1, Hd)
    bi = p["Win"]["b"].reshape(1, -1)
    bo = p["Wout"]["b"].reshape(1, Hd)
    n1g = p["norm1"]["g"].reshape(1, Hd)
    n1b = p["norm1"]["b"].reshape(1, Hd)
    n2g = p["norm2"]["g"].reshape(1, Hd)
    n2b = p["norm2"]["b"].reshape(1, Hd)
    n3g = p["norm3"]["g"].reshape(1, Hd)
    n3b = p["norm3"]["b"].reshape(1, Hd)
    Hi = p["Win"]["w"].shape[1]

    align = _GC * _WORKERS
    ne = _NB * K

    S = 8800
    nb1 = S // _NB
    nb2 = (N - S) // _NB
    M1 = S * K
    idx1 = _pad_idx(idx[:M1], align)
    idx2 = _pad_idx(idx[M1:], align)

    pre1, g1t = pl.pallas_call(
        _proj_body,
        out_shape=[
            jax.ShapeDtypeStruct((N, Hd), jnp.float32),
            jax.ShapeDtypeStruct((N, Hd), jnp.float32),
        ],
    )(hv, w1a, b1, w1c)

    g1a = _sc_gather(g1t, idx1)
    g1b = _sc_gather(g1t, idx2)

    b_consts = [
        _const_spec(Hd, Hd),
        _const_spec(Hd, Hd),
        _const_spec(1, Hd),
        _const_spec(Hd, Hd),
        _const_spec(1, Hd),
        _const_spec(Hd, Hi),
        _const_spec(1, Hi),
        _const_spec(Hi, Hd),
        _const_spec(1, Hd),
        _const_spec(1, Hd),
        _const_spec(1, Hd),
        _const_spec(1, Hd),
        _const_spec(1, Hd),
        _const_spec(Hd, Hd),
        _const_spec(1, Hd),
        _const_spec(Hd, Hd),
    ]
    b_args = (w1b, p["W2"]["w"], b2, p["W3"]["w"], b3,
              p["Win"]["w"], bi, p["Wout"]["w"], bo, n1g, n1b, n2g, n2b,
              w11a, b11, w11c)
    b_out_shape = [
        jax.ShapeDtypeStruct((N, Hd), jnp.float32),
        jax.ShapeDtypeStruct((N, Hd), jnp.float32),
        jax.ShapeDtypeStruct((N, Hd), jnp.float32),
    ]

    hv2a, pre11a, g11a = pl.pallas_call(
        _block1_body,
        grid=(nb1,),
        in_specs=[
            _row_spec(_NB, Hd),
            _row_spec(_NB, Hd),
            _row_spec(ne, Hd),
            _row_spec(ne, Hd),
        ] + b_consts,
        out_specs=[
            _row_spec(_NB, Hd),
            _row_spec(_NB, Hd),
            _row_spec(_NB, Hd),
        ],
        out_shape=b_out_shape,
    )(hv, pre1, he, g1a, *b_args)

    hv2, pre11, g11t = pl.pallas_call(
        _block1_alias,
        grid=(nb2,),
        in_specs=[
            _row_spec(_NB, Hd, nb1),
            _row_spec(_NB, Hd, nb1),
            _row_spec(ne, Hd, nb1),
            _row_spec(ne, Hd),
        ] + b_consts + [
            pl.BlockSpec((8, Hd), lambda i: (0, 0)),
            pl.BlockSpec((8, Hd), lambda i: (0, 0)),
            pl.BlockSpec((8, Hd), lambda i: (0, 0)),
        ],
        out_specs=[
            _row_spec(_NB, Hd, nb1),
            _row_spec(_NB, Hd, nb1),
            _row_spec(_NB, Hd, nb1),
        ],
        out_shape=b_out_shape,
        input_output_aliases={20: 0, 21: 1, 22: 2},
    )(hv, pre1, he, g1b, *b_args, hv2a, pre11a, g11a)

    g2a = _sc_gather(g11t, idx1)
    g2b = _sc_gather(g11t, idx2)

    c_consts = [
        _const_spec(Hd, Hd),
        _const_spec(Hd, Hd),
        _const_spec(1, Hd),
        _const_spec(Hd, Hd),
        _const_spec(1, Hd),
        _const_spec(1, Hd),
        _const_spec(1, Hd),
    ]
    c_args = (w11b, p["W12"]["w"], b12, p["W13"]["w"], b13, n3g, n3b)

    heo1 = pl.pallas_call(
        _block2_body,
        grid=(nb1,),
        in_specs=[
            _row_spec(ne, Hd),
            _row_spec(ne, Hd),
            _row_spec(_NB, Hd),
        ] + c_consts,
        out_specs=[_row_spec(ne, Hd)],
        out_shape=[jax.ShapeDtypeStruct((M, Hd), jnp.float32)],
    )(he, g2a, pre11, *c_args)[0]

    heo = pl.pallas_call(
        _block2_alias,
        grid=(nb2,),
        in_specs=[
            _row_spec(ne, Hd, nb1),
            _row_spec(ne, Hd),
            _row_spec(_NB, Hd, nb1),
        ] + c_consts + [
            pl.BlockSpec((8, Hd), lambda i: (0, 0)),
        ],
        out_specs=[_row_spec(ne, Hd, nb1)],
        out_shape=[jax.ShapeDtypeStruct((M, Hd), jnp.float32)],
        input_output_aliases={10: 0},
    )(he, g2b, pre11, *c_args, heo1)[0]

    return hv2.reshape(B, N, Hd), heo.reshape(B, N, K, Hd)
